# baseline (speedup 1.0000x reference)
import functools

import jax
import jax.numpy as jnp
from jax import lax
from jax.experimental import pallas as pl
from jax.experimental.pallas import tpu as pltpu
from jax.experimental.pallas import tpu_sc as plsc

BATCH = 4096
SEQ = 200
DIM = 64
NUM_CORES = 2
NUM_SUBCORES = 16
NW = NUM_CORES * NUM_SUBCORES
BPW = BATCH // NW
SPLIT_A = 104
SPLIT_B = SEQ - SPLIT_A


def _body(x_tok, y_tok, lane_idx_hbm, x_tab, y_tab, p_tab, l_tab, out_hbm,
          xt_ring, yt_ring, lane_idx, lane_rows, pos_v, buf_x, buf_y, obuf,
          sem_x0, sem_x1, sem_y0, sem_y1, sem_o0, sem_o1,
          sem_t0, sem_t1, sem_t2, sem_t3):
    cid = lax.axis_index("c")
    sid = lax.axis_index("s")
    wid = sid * NUM_CORES + cid
    base_b = wid * BPW

    sem_x = [sem_x0, sem_x1]
    sem_y = [sem_y0, sem_y1]
    sem_o = [sem_o0, sem_o1]
    sem_t = [sem_t0, sem_t1, sem_t2, sem_t3]

    pltpu.sync_copy(p_tab, pos_v)
    pltpu.sync_copy(lane_idx_hbm.at[pl.ds(base_b, BPW)], lane_idx)
    pltpu.async_copy(l_tab.at[lane_idx], lane_rows, sem_x0).wait()

    def tok_descs(j, t):
        off = pl.multiple_of((base_b + j) * SEQ, 8)
        return (
            pltpu.make_async_copy(x_tok.at[pl.ds(off, SEQ)],
                                  xt_ring.at[t], sem_t[t]),
            pltpu.make_async_copy(y_tok.at[pl.ds(off, SEQ)],
                                  yt_ring.at[t], sem_t[t]),
        )

    def gather_descs(t, p):
        return (
            pltpu.make_async_copy(x_tab.at[xt_ring.at[t, pl.ds(0, SPLIT_A)]],
                                  buf_x.at[p, pl.ds(0, SPLIT_A)], sem_x[p]),
            pltpu.make_async_copy(
                x_tab.at[xt_ring.at[t, pl.ds(SPLIT_A, SPLIT_B)]],
                buf_x.at[p, pl.ds(SPLIT_A, SPLIT_B)], sem_x[p]),
            pltpu.make_async_copy(y_tab.at[yt_ring.at[t, pl.ds(0, SPLIT_A)]],
                                  buf_y.at[p, pl.ds(0, SPLIT_A)], sem_y[p]),
            pltpu.make_async_copy(
                y_tab.at[yt_ring.at[t, pl.ds(SPLIT_A, SPLIT_B)]],
                buf_y.at[p, pl.ds(SPLIT_A, SPLIT_B)], sem_y[p]),
        )

    def out_desc(j, p):
        return pltpu.make_async_copy(obuf.at[p], out_hbm.at[base_b + j],
                                     sem_o[p])

    for t in range(4):
        for d in tok_descs(t, t):
            d.start()
    for t in range(2):
        sem = sem_t[t]
        for d in tok_descs(t, t):
            d.wait()
        for d in gather_descs(t, t):
            d.start()

    def batch_body(j, carry):
        p = lax.rem(j, 2)

        for ps in range(2):
            @pl.when(p == ps)
            def _():
                for d in gather_descs(0, ps):
                    d.wait()

                @pl.when(j >= 2)
                def _():
                    out_desc(j - 2, ps).wait()

                lane_vecs = [lane_rows[j, pl.ds(q * 16, 16)] * 10.0
                             for q in range(4)]

                @plsc.parallel_loop(0, SEQ, unroll=4)
                def _(r):
                    for q in range(4):
                        sl = pl.ds(q * 16, 16)
                        obuf[ps, r, sl] = (buf_x[ps, r, sl] + buf_y[ps, r, sl]
                                           + pos_v[r, sl] + lane_vecs[q])

                out_desc(j, ps).start()

                @pl.when(j + 2 < BPW)
                def _():
                    t2 = lax.rem(j + 2, 4)
                    for ts in range(4):
                        @pl.when(t2 == ts)
                        def _():
                            for d in tok_descs(j + 2, ts):
                                d.wait()
                            for d in gather_descs(ts, ps):
                                d.start()

                @pl.when(j + 4 < BPW)
                def _():
                    t4 = lax.rem(j + 4, 4)
                    for ts in range(4):
                        @pl.when(t4 == ts)
                        def _():
                            for d in tok_descs(j + 4, ts):
                                d.start()
        return carry

    lax.fori_loop(0, BPW, batch_body, 0)

    out_desc(BPW - 2, 0).wait()
    out_desc(BPW - 1, 1).wait()


_sc_call = functools.partial(
    pl.kernel,
    mesh=plsc.VectorSubcoreMesh(core_axis_name="c", subcore_axis_name="s"),
    out_type=jax.ShapeDtypeStruct((BATCH, SEQ, DIM), jnp.float32),
    scratch_types=[
        pltpu.VMEM((4, SEQ), jnp.int32),
        pltpu.VMEM((4, SEQ), jnp.int32),
        pltpu.VMEM((BPW,), jnp.int32),
        pltpu.VMEM((BPW, DIM), jnp.float32),
        pltpu.VMEM((SEQ, DIM), jnp.float32),
        pltpu.VMEM((2, SEQ, DIM), jnp.float32),
        pltpu.VMEM((2, SEQ, DIM), jnp.float32),
        pltpu.VMEM((2, SEQ, DIM), jnp.float32),
        pltpu.SemaphoreType.DMA,
        pltpu.SemaphoreType.DMA,
        pltpu.SemaphoreType.DMA,
        pltpu.SemaphoreType.DMA,
        pltpu.SemaphoreType.DMA,
        pltpu.SemaphoreType.DMA,
        pltpu.SemaphoreType.DMA,
        pltpu.SemaphoreType.DMA,
        pltpu.SemaphoreType.DMA,
        pltpu.SemaphoreType.DMA,
    ],
    compiler_params=pltpu.CompilerParams(use_tc_tiling_on_sc=False),
)(_body)


@jax.jit
def kernel(x_tokens, y_tokens, lane_indices, x_table, y_table, pos_table,
           lane_table):
    x_tokens = x_tokens.astype(jnp.int32).reshape(BATCH * SEQ)
    y_tokens = y_tokens.astype(jnp.int32).reshape(BATCH * SEQ)
    lane_indices = lane_indices.astype(jnp.int32)
    return _sc_call(x_tokens, y_tokens, lane_indices, x_table, y_table,
                    pos_table, lane_table)


# speedup vs baseline: 1.0001x; 1.0001x over previous
import functools

import jax
import jax.numpy as jnp
from jax import lax
from jax.experimental import pallas as pl
from jax.experimental.pallas import tpu as pltpu
from jax.experimental.pallas import tpu_sc as plsc

BATCH = 4096
SEQ = 200
DIM = 64
NUM_CORES = 2
NUM_SUBCORES = 16
NW = NUM_CORES * NUM_SUBCORES
BPW = BATCH // NW


def _body(x_tok, y_tok, lane_idx_hbm, x_tab, y_tab, p_tab, l_tab, out_hbm,
          xt_ring, yt_ring, lane_idx, lane_rows, pos_v, buf_y, obuf,
          sem_x0, sem_x1, sem_y0, sem_y1,
          sem_o0, sem_o1, sem_o2, sem_o3,
          sem_t0, sem_t1, sem_t2, sem_t3):
    cid = lax.axis_index("c")
    sid = lax.axis_index("s")
    wid = sid * NUM_CORES + cid
    base_b = wid * BPW

    sem_x = [sem_x0, sem_x1]
    sem_y = [sem_y0, sem_y1]
    sem_o = [sem_o0, sem_o1, sem_o2, sem_o3]
    sem_t = [sem_t0, sem_t1, sem_t2, sem_t3]

    pltpu.sync_copy(p_tab, pos_v)
    pltpu.sync_copy(lane_idx_hbm.at[pl.ds(base_b, BPW)], lane_idx)
    pltpu.async_copy(l_tab.at[lane_idx], lane_rows, sem_x0).wait()

    def tok_descs(j, t):
        off = pl.multiple_of((base_b + j) * SEQ, 8)
        return (
            pltpu.make_async_copy(x_tok.at[pl.ds(off, SEQ)],
                                  xt_ring.at[t], sem_t[t]),
            pltpu.make_async_copy(y_tok.at[pl.ds(off, SEQ)],
                                  yt_ring.at[t], sem_t[t]),
        )

    def x_desc(t, p4):
        return pltpu.make_async_copy(x_tab.at[xt_ring.at[t]],
                                     obuf.at[p4], sem_x[p4 % 2])

    def y_desc(t, p2):
        return pltpu.make_async_copy(y_tab.at[yt_ring.at[t]],
                                     buf_y.at[p2], sem_y[p2])

    def out_desc(j, p4):
        return pltpu.make_async_copy(obuf.at[p4], out_hbm.at[base_b + j],
                                     sem_o[p4])

    for t in range(4):
        for d in tok_descs(t, t):
            d.start()
    for t in range(2):
        for d in tok_descs(t, t):
            d.wait()
        x_desc(t, t).start()
        y_desc(t, t).start()

    def batch_body(j, carry):
        jm4 = lax.rem(j, 4)
        for ps in range(4):
            @pl.when(jm4 == ps)
            def _():
                p2 = ps % 2
                x_desc(0, ps).wait()
                y_desc(0, p2).wait()

                lane_vecs = [lane_rows[j, pl.ds(q * 16, 16)] * 10.0
                             for q in range(4)]

                @plsc.parallel_loop(0, SEQ, unroll=2)
                def _(r):
                    for q in range(4):
                        sl = pl.ds(q * 16, 16)
                        plsc.addupdate(
                            obuf.at[ps, r, sl],
                            buf_y[p2, r, sl] + pos_v[r, sl] + lane_vecs[q])

                out_desc(j, ps).start()

                ns = (ps + 2) % 4
                @pl.when(j + 2 < BPW)
                def _():
                    for d in tok_descs(j + 2, ns):
                        d.wait()

                    @pl.when(j >= 2)
                    def _():
                        out_desc(j - 2, ns).wait()

                    x_desc(ns, ns).start()
                    y_desc(ns, p2).start()

                @pl.when(j + 4 < BPW)
                def _():
                    for d in tok_descs(j + 4, ps):
                        d.start()
        return carry

    lax.fori_loop(0, BPW, batch_body, 0)

    for j in range(BPW - 4, BPW):
        out_desc(j, j % 4).wait()


_sc_call = functools.partial(
    pl.kernel,
    mesh=plsc.VectorSubcoreMesh(core_axis_name="c", subcore_axis_name="s"),
    out_type=jax.ShapeDtypeStruct((BATCH, SEQ, DIM), jnp.float32),
    scratch_types=[
        pltpu.VMEM((4, SEQ), jnp.int32),
        pltpu.VMEM((4, SEQ), jnp.int32),
        pltpu.VMEM((BPW,), jnp.int32),
        pltpu.VMEM((BPW, DIM), jnp.float32),
        pltpu.VMEM((SEQ, DIM), jnp.float32),
        pltpu.VMEM((2, SEQ, DIM), jnp.float32),
        pltpu.VMEM((4, SEQ, DIM), jnp.float32),
        pltpu.SemaphoreType.DMA,
        pltpu.SemaphoreType.DMA,
        pltpu.SemaphoreType.DMA,
        pltpu.SemaphoreType.DMA,
        pltpu.SemaphoreType.DMA,
        pltpu.SemaphoreType.DMA,
        pltpu.SemaphoreType.DMA,
        pltpu.SemaphoreType.DMA,
        pltpu.SemaphoreType.DMA,
        pltpu.SemaphoreType.DMA,
        pltpu.SemaphoreType.DMA,
        pltpu.SemaphoreType.DMA,
    ],
    compiler_params=pltpu.CompilerParams(use_tc_tiling_on_sc=False),
)(_body)


@jax.jit
def kernel(x_tokens, y_tokens, lane_indices, x_table, y_table, pos_table,
           lane_table):
    x_tokens = x_tokens.astype(jnp.int32).reshape(BATCH * SEQ)
    y_tokens = y_tokens.astype(jnp.int32).reshape(BATCH * SEQ)
    lane_indices = lane_indices.astype(jnp.int32)
    return _sc_call(x_tokens, y_tokens, lane_indices, x_table, y_table,
                    pos_table, lane_table)


# speedup vs baseline: 1.7848x; 1.7846x over previous
import functools

import jax
import jax.numpy as jnp
from jax import lax
from jax.experimental import pallas as pl
from jax.experimental.pallas import tpu as pltpu
from jax.experimental.pallas import tpu_sc as plsc

BATCH = 4096
SEQ = 200
DIM = 64
NUM_CORES = 2
NUM_SUBCORES = 16
NW = NUM_CORES * NUM_SUBCORES
BPW = BATCH // NW
XPAD = 1024
YSTRIDE = 208
XPW = XPAD // NW
TROWS = XPAD * YSTRIDE


def _build_body(x_pad, y_pad, t_xy, xchunk, ybuf, bbuf, sem_b0, sem_b1):
    cid = lax.axis_index("c")
    sid = lax.axis_index("s")
    wid = sid * NUM_CORES + cid

    sem_b = [sem_b0, sem_b1]
    pltpu.sync_copy(x_pad.at[pl.ds(wid * XPW, XPW)], xchunk)
    pltpu.sync_copy(y_pad, ybuf)

    def out_desc(k, slot):
        row0 = pl.multiple_of((wid * XPW + k) * YSTRIDE, 8)
        return pltpu.make_async_copy(bbuf.at[slot],
                                     t_xy.at[pl.ds(row0, YSTRIDE)],
                                     sem_b[slot])

    def k_body(k, carry):
        xv = [xchunk[k, pl.ds(q * 16, 16)] for q in range(4)]
        for slot in range(2):
            @pl.when(lax.rem(k, 2) == slot)
            def _():
                @pl.when(k >= 2)
                def _():
                    out_desc(k - 2, slot).wait()

                @plsc.parallel_loop(0, YSTRIDE, unroll=2)
                def _(yr):
                    for q in range(4):
                        sl = pl.ds(q * 16, 16)
                        bbuf[slot, yr, sl] = ybuf[yr, sl] + xv[q]

                out_desc(k, slot).start()
        return carry

    lax.fori_loop(0, XPW, k_body, 0)
    out_desc(XPW - 2, 0).wait()
    out_desc(XPW - 1, 1).wait()


_build_call = functools.partial(
    pl.kernel,
    mesh=plsc.VectorSubcoreMesh(core_axis_name="c", subcore_axis_name="s"),
    out_type=jax.ShapeDtypeStruct((TROWS, DIM), jnp.float32),
    scratch_types=[
        pltpu.VMEM((XPW, DIM), jnp.float32),
        pltpu.VMEM((YSTRIDE, DIM), jnp.float32),
        pltpu.VMEM((2, YSTRIDE, DIM), jnp.float32),
        pltpu.SemaphoreType.DMA,
        pltpu.SemaphoreType.DMA,
    ],
    compiler_params=pltpu.CompilerParams(use_tc_tiling_on_sc=False),
)(_build_body)


def _main_body(idx_flat, lane_idx_hbm, t_xy, p_tab, l_tab, out_hbm,
               it_ring, lane_idx, lane_rows, pos_v, obuf,
               sem_x0, sem_x1,
               sem_o0, sem_o1, sem_o2, sem_o3,
               sem_t0, sem_t1, sem_t2, sem_t3):
    cid = lax.axis_index("c")
    sid = lax.axis_index("s")
    wid = sid * NUM_CORES + cid
    base_b = wid * BPW

    sem_x = [sem_x0, sem_x1]
    sem_o = [sem_o0, sem_o1, sem_o2, sem_o3]
    sem_t = [sem_t0, sem_t1, sem_t2, sem_t3]

    pltpu.sync_copy(p_tab, pos_v)
    pltpu.sync_copy(lane_idx_hbm.at[pl.ds(base_b, BPW)], lane_idx)
    pltpu.async_copy(l_tab.at[lane_idx], lane_rows, sem_x0).wait()

    def tok_desc(j, t):
        off = pl.multiple_of((base_b + j) * SEQ, 8)
        return pltpu.make_async_copy(idx_flat.at[pl.ds(off, SEQ)],
                                     it_ring.at[t], sem_t[t])

    def x_desc(t, p4):
        return pltpu.make_async_copy(t_xy.at[it_ring.at[t]],
                                     obuf.at[p4], sem_x[p4 % 2])

    def out_desc(j, p4):
        return pltpu.make_async_copy(obuf.at[p4], out_hbm.at[base_b + j],
                                     sem_o[p4])

    for t in range(4):
        tok_desc(t, t).start()
    for t in range(2):
        tok_desc(t, t).wait()
        x_desc(t, t).start()

    def batch_body(j, carry):
        jm4 = lax.rem(j, 4)
        for ps in range(4):
            @pl.when(jm4 == ps)
            def _():
                x_desc(0, ps).wait()

                lane_vecs = [lane_rows[j, pl.ds(q * 16, 16)] * 10.0
                             for q in range(4)]

                @plsc.parallel_loop(0, SEQ, unroll=2)
                def _(r):
                    for q in range(4):
                        sl = pl.ds(q * 16, 16)
                        plsc.addupdate(obuf.at[ps, r, sl],
                                       pos_v[r, sl] + lane_vecs[q])

                out_desc(j, ps).start()

                ns = (ps + 2) % 4
                @pl.when(j + 2 < BPW)
                def _():
                    tok_desc(j + 2, ns).wait()

                    @pl.when(j >= 2)
                    def _():
                        out_desc(j - 2, ns).wait()

                    x_desc(ns, ns).start()

                @pl.when(j + 4 < BPW)
                def _():
                    tok_desc(j + 4, ps).start()
        return carry

    lax.fori_loop(0, BPW, batch_body, 0)

    for j in range(BPW - 4, BPW):
        out_desc(j, j % 4).wait()


_main_call = functools.partial(
    pl.kernel,
    mesh=plsc.VectorSubcoreMesh(core_axis_name="c", subcore_axis_name="s"),
    out_type=jax.ShapeDtypeStruct((BATCH, SEQ, DIM), jnp.float32),
    scratch_types=[
        pltpu.VMEM((4, SEQ), jnp.int32),
        pltpu.VMEM((BPW,), jnp.int32),
        pltpu.VMEM((BPW, DIM), jnp.float32),
        pltpu.VMEM((SEQ, DIM), jnp.float32),
        pltpu.VMEM((4, SEQ, DIM), jnp.float32),
        pltpu.SemaphoreType.DMA,
        pltpu.SemaphoreType.DMA,
        pltpu.SemaphoreType.DMA,
        pltpu.SemaphoreType.DMA,
        pltpu.SemaphoreType.DMA,
        pltpu.SemaphoreType.DMA,
        pltpu.SemaphoreType.DMA,
        pltpu.SemaphoreType.DMA,
        pltpu.SemaphoreType.DMA,
        pltpu.SemaphoreType.DMA,
    ],
    compiler_params=pltpu.CompilerParams(use_tc_tiling_on_sc=False),
)(_main_body)


@jax.jit
def kernel(x_tokens, y_tokens, lane_indices, x_table, y_table, pos_table,
           lane_table):
    x_tokens = x_tokens.astype(jnp.int32)
    y_tokens = y_tokens.astype(jnp.int32)
    lane_indices = lane_indices.astype(jnp.int32)
    idx_flat = (x_tokens * YSTRIDE + y_tokens).reshape(BATCH * SEQ)
    x_pad = jnp.pad(x_table, ((0, XPAD - x_table.shape[0]), (0, 0)))
    y_pad = jnp.pad(y_table, ((0, YSTRIDE - y_table.shape[0]), (0, 0)))
    t_xy = _build_call(x_pad, y_pad)
    return _main_call(idx_flat, lane_indices, t_xy, pos_table, lane_table)
